# baseline (speedup 1.0000x reference)
import functools

import jax
import jax.numpy as jnp
from jax import lax
from jax.experimental import pallas as pl
from jax.experimental.pallas import tpu as pltpu
from jax.experimental.pallas import tpu_sc as plsc

N = 10000
D = 128
E = 320000
NC, NS = 2, 16
NW = NC * NS
W = 96
NBUF = 3
NWIN = 108
EPT = NWIN * W
EPAD = NW * EPT
RPT = 632
NPAD = RPT * NS
NPR = NPAD - N


def _segsum_body(h_hbm, src_hbm, dst_hbm, out_hbm, *refs):
    srcs = (refs[0:NBUF], refs[NBUF:2 * NBUF])
    dsts = (refs[2 * NBUF:3 * NBUF], refs[3 * NBUF:4 * NBUF])
    rows = refs[4 * NBUF:5 * NBUF]
    agg_sh = refs[5 * NBUF]
    isems = (refs[5 * NBUF + 1:6 * NBUF + 1],
             refs[6 * NBUF + 1:7 * NBUF + 1])
    gsems = refs[7 * NBUF + 1:8 * NBUF + 1]

    cid = lax.axis_index("c")
    sid = lax.axis_index("s")
    wid = cid * NS + sid

    zero = jnp.zeros((16,), jnp.float32)

    def zstore(k, _):
        rows[0][k // (D // 16), pl.ds((k % (D // 16)) * 16, 16)] = zero
        return 0
    lax.fori_loop(0, W * (D // 16), zstore, 0)

    rbase = sid * RPT

    def zcopy(k, _):
        pltpu.sync_copy(rows[0], agg_sh.at[pl.ds(rbase + k * W, W)])
        return 0
    lax.fori_loop(0, RPT // W, zcopy, 0)
    pltpu.sync_copy(rows[0].at[pl.ds(0, RPT - (RPT // W) * W)],
                    agg_sh.at[pl.ds(rbase + (RPT // W) * W,
                                    RPT - (RPT // W) * W)])
    plsc.subcore_barrier()

    def pk_start(w, b, p):
        pltpu.async_copy(src_hbm.at[wid, w], srcs[p][b], isems[p][b])
        pltpu.async_copy(dst_hbm.at[wid, w], dsts[p][b], isems[p][b])

    def pk_wait(w, b, p):
        pltpu.make_async_copy(src_hbm.at[wid, w], srcs[p][b],
                              isems[p][b]).wait()
        pltpu.make_async_copy(dst_hbm.at[wid, w], dsts[p][b],
                              isems[p][b]).wait()

    def g_start(b, p):
        pltpu.async_copy(h_hbm.at[srcs[p][b]], rows[b], gsems[b])

    def g_wait(b, p):
        pltpu.make_async_copy(h_hbm.at[srcs[p][b]], rows[b],
                              gsems[b]).wait()

    for b in range(NBUF):
        pk_start(b, b, 0)
        pk_start(NBUF + b, b, 1)
    for b in range(NBUF):
        pk_wait(b, b, 0)
        g_start(b, 0)

    NR = NWIN // NBUF

    def body(i, _):
        for p in (0, 1):
            r = 2 * i + p
            for b in range(NBUF):
                w = r * NBUF + b
                g_wait(b, p)
                pltpu.sync_copy(rows[b], agg_sh.at[dsts[p][b]], add=True)

                @pl.when(r < NR - 1)
                def _(b=b, w=w, p=p):
                    pk_wait(w + NBUF, b, 1 - p)
                    g_start(b, 1 - p)

                    @pl.when(r < NR - 2)
                    def _(b=b, w=w, p=p):
                        pk_start(w + 2 * NBUF, b, p)
        return 0
    lax.fori_loop(0, NR // 2, body, 0)
    plsc.subcore_barrier()

    pltpu.sync_copy(agg_sh.at[pl.ds(rbase, RPT)],
                    out_hbm.at[cid, pl.ds(rbase, RPT)])


_segsum = pl.kernel(
    _segsum_body,
    out_type=jax.ShapeDtypeStruct((NC, NPAD, D), jnp.float32),
    mesh=plsc.VectorSubcoreMesh(core_axis_name="c", subcore_axis_name="s"),
    scratch_types=(
        [pltpu.VMEM((W,), jnp.int32) for _ in range(4 * NBUF)]
        + [pltpu.VMEM((W, D), jnp.float32) for _ in range(NBUF)]
        + [pltpu.VMEM_SHARED((NPAD, D), jnp.float32)]
        + [pltpu.SemaphoreType.DMA for _ in range(3 * NBUF)]
    ),
)


def _layer_tc_body(parts_ref, h_ref, wr_ref, wroot_ref, b_ref, o_ref,
                   *, relu, residual):
    agg = parts_ref[0] + parts_ref[1]
    acc = jnp.dot(agg, wr_ref[...], preferred_element_type=jnp.float32)
    acc = acc + jnp.dot(h_ref[...], wroot_ref[...],
                        preferred_element_type=jnp.float32)
    acc = acc + b_ref[...]
    if relu:
        acc = jnp.maximum(acc, 0.0)
    if residual:
        acc = acc + h_ref[...]
    o_ref[...] = acc


BM = 1264


def _layer_tc(parts, h, wr, wroot, b, relu, residual):
    body = functools.partial(_layer_tc_body, relu=relu, residual=residual)
    return pl.pallas_call(
        body,
        grid=(NPAD // BM,),
        in_specs=[
            pl.BlockSpec((NC, BM, D), lambda i: (0, i, 0)),
            pl.BlockSpec((BM, D), lambda i: (i, 0)),
            pl.BlockSpec((D, D), lambda i: (0, 0)),
            pl.BlockSpec((D, D), lambda i: (0, 0)),
            pl.BlockSpec((1, D), lambda i: (0, 0)),
        ],
        out_specs=pl.BlockSpec((BM, D), lambda i: (i, 0)),
        out_shape=jax.ShapeDtypeStruct((NPAD, D), jnp.float32),
    )(parts, h, wr, wroot, b)


def kernel(x, edge_index, Wr, br, Wroot, Wpos_rel, bpos, Wpos_root):
    npad_e = EPAD - E
    pad_idx = (N + (jnp.arange(npad_e, dtype=jnp.int32) % NPR))
    src = jnp.concatenate([edge_index[0], pad_idx]).reshape(NW, NWIN, W)
    dst = jnp.concatenate([edge_index[1], pad_idx]).reshape(NW, NWIN, W)

    h = jnp.zeros((NPAD, D), jnp.float32).at[:N].set(x)
    for i in range(12):
        parts = _segsum(h, src, dst)
        h = _layer_tc(parts, h, Wr[i], Wroot[i], br[i].reshape(1, D),
                      relu=True, residual=(i > 0))

    parts = _segsum(h, src, dst)
    wpr = jnp.zeros((D, D), jnp.float32).at[:, :3].set(Wpos_rel)
    wpt = jnp.zeros((D, D), jnp.float32).at[:, :3].set(Wpos_root)
    bp = jnp.zeros((1, D), jnp.float32).at[0, :3].set(bpos)
    pos = _layer_tc(parts, h, wpr, wpt, bp, relu=False, residual=False)
    return h[:N], pos[:N, :3]


# speedup vs baseline: 1.0485x; 1.0485x over previous
import functools

import jax
import jax.numpy as jnp
from jax import lax
from jax.experimental import pallas as pl
from jax.experimental.pallas import tpu as pltpu
from jax.experimental.pallas import tpu_sc as plsc

N = 10000
D = 128
E = 320000
NC, NS = 2, 16
NW = NC * NS
W = 64
NBUF = 4
NWIN = 160
EPT = NWIN * W
EPAD = NW * EPT
RPT = 632
NPAD = RPT * NS
NPR = NPAD - N


def _segsum_body(h_hbm, pk_hbm, out_hbm, *refs):
    pkbs = refs[0:NBUF]
    srcs = refs[NBUF:2 * NBUF]
    dsts = refs[2 * NBUF:3 * NBUF]
    rows = refs[3 * NBUF:4 * NBUF]
    agg_sh = refs[4 * NBUF]
    isems = refs[4 * NBUF + 1:5 * NBUF + 1]
    gsems = refs[5 * NBUF + 1:6 * NBUF + 1]

    cid = lax.axis_index("c")
    sid = lax.axis_index("s")
    wid = cid * NS + sid

    zero = jnp.zeros((16,), jnp.float32)

    def zstore(k, _):
        rows[0][k // (D // 16), pl.ds((k % (D // 16)) * 16, 16)] = zero
        return 0
    lax.fori_loop(0, W * (D // 16), zstore, 0)

    rbase = sid * RPT

    def zcopy(k, _):
        pltpu.sync_copy(rows[0], agg_sh.at[pl.ds(rbase + k * W, W)])
        return 0
    lax.fori_loop(0, RPT // W, zcopy, 0)
    pltpu.sync_copy(rows[0].at[pl.ds(0, RPT - (RPT // W) * W)],
                    agg_sh.at[pl.ds(rbase + (RPT // W) * W,
                                    RPT - (RPT // W) * W)])
    plsc.subcore_barrier()

    def unpack(pk_b, src_b, dst_b):
        def u(k, _):
            v = pk_b[pl.ds(k * 16, 16)]
            src_b[pl.ds(k * 16, 16)] = jnp.bitwise_and(v, 0xFFFF)
            dst_b[pl.ds(k * 16, 16)] = lax.shift_right_logical(v, 16)
            return 0
        lax.fori_loop(0, W // 16, u, 0)

    def pk_start(w, b):
        pltpu.async_copy(pk_hbm.at[wid, w], pkbs[b], isems[b])

    def pk_wait(w, b):
        pltpu.make_async_copy(pk_hbm.at[wid, w], pkbs[b], isems[b]).wait()

    def g_start(b):
        pltpu.async_copy(h_hbm.at[srcs[b]], rows[b], gsems[b])

    def g_wait(b):
        pltpu.make_async_copy(h_hbm.at[srcs[b]], rows[b], gsems[b]).wait()

    for b in range(NBUF):
        pk_start(b, b)
    for b in range(NBUF):
        pk_wait(b, b)
        unpack(pkbs[b], srcs[b], dsts[b])
        g_start(b)
        pk_start(b + NBUF, b)

    def body(i, _):
        for b in range(NBUF):
            w = i * NBUF + b
            g_wait(b)
            pltpu.sync_copy(rows[b], agg_sh.at[dsts[b]], add=True)

            @pl.when(i < NWIN // NBUF - 1)
            def _(b=b, w=w):
                pk_wait(w + NBUF, b)
                unpack(pkbs[b], srcs[b], dsts[b])
                g_start(b)

                @pl.when(i < NWIN // NBUF - 2)
                def _(b=b, w=w):
                    pk_start(w + 2 * NBUF, b)
        return 0
    lax.fori_loop(0, NWIN // NBUF, body, 0)
    plsc.subcore_barrier()

    pltpu.sync_copy(agg_sh.at[pl.ds(rbase, RPT)],
                    out_hbm.at[cid, pl.ds(rbase, RPT)])


_segsum = pl.kernel(
    _segsum_body,
    out_type=jax.ShapeDtypeStruct((NC, NPAD, D), jnp.float32),
    mesh=plsc.VectorSubcoreMesh(core_axis_name="c", subcore_axis_name="s"),
    scratch_types=(
        [pltpu.VMEM((W,), jnp.int32) for _ in range(3 * NBUF)]
        + [pltpu.VMEM((W, D), jnp.float32) for _ in range(NBUF)]
        + [pltpu.VMEM_SHARED((NPAD, D), jnp.float32)]
        + [pltpu.SemaphoreType.DMA for _ in range(2 * NBUF)]
    ),
)


def _layer_tc_body(parts_ref, h_ref, wr_ref, wroot_ref, b_ref, o_ref,
                   *, relu, residual):
    agg = parts_ref[0] + parts_ref[1]
    acc = jnp.dot(agg, wr_ref[...], preferred_element_type=jnp.float32)
    acc = acc + jnp.dot(h_ref[...], wroot_ref[...],
                        preferred_element_type=jnp.float32)
    acc = acc + b_ref[...]
    if relu:
        acc = jnp.maximum(acc, 0.0)
    if residual:
        acc = acc + h_ref[...]
    o_ref[...] = acc


BM = 1264


def _layer_tc(parts, h, wr, wroot, b, relu, residual):
    body = functools.partial(_layer_tc_body, relu=relu, residual=residual)
    return pl.pallas_call(
        body,
        grid=(NPAD // BM,),
        in_specs=[
            pl.BlockSpec((NC, BM, D), lambda i: (0, i, 0)),
            pl.BlockSpec((BM, D), lambda i: (i, 0)),
            pl.BlockSpec((D, D), lambda i: (0, 0)),
            pl.BlockSpec((D, D), lambda i: (0, 0)),
            pl.BlockSpec((1, D), lambda i: (0, 0)),
        ],
        out_specs=pl.BlockSpec((BM, D), lambda i: (i, 0)),
        out_shape=jax.ShapeDtypeStruct((NPAD, D), jnp.float32),
    )(parts, h, wr, wroot, b)


def kernel(x, edge_index, Wr, br, Wroot, Wpos_rel, bpos, Wpos_root):
    npad_e = EPAD - E
    pad_idx = (N + (jnp.arange(npad_e, dtype=jnp.int32) % NPR))
    src = jnp.concatenate([edge_index[0], pad_idx])
    dst = jnp.concatenate([edge_index[1], pad_idx])
    packed = (src | (dst << 16)).reshape(NW, NWIN, W)

    h = jnp.zeros((NPAD, D), jnp.float32).at[:N].set(x)
    for i in range(12):
        parts = _segsum(h, packed)
        h = _layer_tc(parts, h, Wr[i], Wroot[i], br[i].reshape(1, D),
                      relu=True, residual=(i > 0))

    parts = _segsum(h, packed)
    wpr = jnp.zeros((D, D), jnp.float32).at[:, :3].set(Wpos_rel)
    wpt = jnp.zeros((D, D), jnp.float32).at[:, :3].set(Wpos_root)
    bp = jnp.zeros((1, D), jnp.float32).at[0, :3].set(bpos)
    pos = _layer_tc(parts, h, wpr, wpt, bp, relu=False, residual=False)
    return h[:N], pos[:N, :3]
